# Initial kernel scaffold; baseline (speedup 1.0000x reference)
import jax
import jax.numpy as jnp
from jax.experimental import pallas as pl


def kernel(x, seg_idx, seg_embed):
    raise NotImplementedError("write your pallas kernel here")


# speedup vs baseline: 3.9132x; 3.9132x over previous
import functools

import jax
import jax.numpy as jnp
from jax import lax
from jax.experimental import pallas as pl
from jax.experimental.pallas import tpu as pltpu
from jax.experimental.pallas import tpu_sc as plsc

D = 128
LANES = 16
NUM_CORES = 2
NUM_SUBCORES = 16
NUM_WORKERS = NUM_CORES * NUM_SUBCORES
CHUNK = 256
GATHER_GROUP = 128


def _sc_body(n_tokens, vocab, x_hbm, idx_hbm, tab_hbm, out_hbm,
             idx_v, xrow, trow, tab_sh, sem):
  per_w = n_tokens // NUM_WORKERS
  chunks = per_w // CHUNK
  cid = lax.axis_index("c")
  sid = lax.axis_index("s")
  wid = sid * NUM_CORES + cid

  @pl.when(sid == 0)
  def _stage():
    pltpu.sync_copy(tab_hbm, trow.at[pl.ds(0, vocab)])
    for j in range(D // LANES):
      trow[0, pl.ds(j * LANES, LANES)] = jnp.zeros((LANES,), jnp.float32)
    pltpu.sync_copy(trow.at[pl.ds(0, vocab)], tab_sh)

  plsc.subcore_barrier()

  def chunk_body(t, carry):
    base = wid * per_w + t * CHUNK
    pltpu.sync_copy(idx_hbm.at[pl.ds(base, CHUNK)], idx_v)
    pltpu.sync_copy(x_hbm.at[pl.ds(base, CHUNK)], xrow)
    cps = [
        pltpu.async_copy(
            tab_sh.at[idx_v.at[pl.ds(g * GATHER_GROUP, GATHER_GROUP)]],
            trow.at[pl.ds(g * GATHER_GROUP, GATHER_GROUP)],
            sem,
        )
        for g in range(CHUNK // GATHER_GROUP)
    ]
    for cp in cps:
      cp.wait()

    def tok_body(i, c):
      for j in range(D // LANES):
        s = pl.ds(j * LANES, LANES)
        xrow[i, s] = xrow[i, s] + trow[i, s]
      return c

    lax.fori_loop(0, CHUNK, tok_body, 0)
    pltpu.sync_copy(xrow, out_hbm.at[pl.ds(base, CHUNK)])
    return carry

  lax.fori_loop(0, chunks, chunk_body, 0)


@functools.partial(jax.jit, static_argnames=())
def kernel(x, seg_idx, seg_embed):
  b, s, d = x.shape
  n = b * s
  vocab = seg_embed.shape[0]
  xf = x.reshape(n, d)
  idxf = seg_idx.reshape(n).astype(jnp.int32)
  tab = seg_embed.astype(jnp.float32)

  mesh = plsc.VectorSubcoreMesh(
      core_axis_name="c", subcore_axis_name="s",
      num_cores=NUM_CORES, num_subcores=NUM_SUBCORES,
  )
  out = pl.kernel(
      functools.partial(_sc_body, n, vocab),
      out_type=jax.ShapeDtypeStruct((n, d), jnp.float32),
      mesh=mesh,
      scratch_types=[
          pltpu.VMEM((CHUNK,), jnp.int32),
          pltpu.VMEM((CHUNK, D), jnp.float32),
          pltpu.VMEM((CHUNK, D), jnp.float32),
          pltpu.VMEM_SHARED((vocab, D), jnp.float32),
          pltpu.SemaphoreType.DMA,
      ],
  )(xf, idxf, tab)
  return out.reshape(b, s, d)


# speedup vs baseline: 7.4842x; 1.9126x over previous
import functools

import jax
import jax.numpy as jnp
from jax import lax
from jax.experimental import pallas as pl
from jax.experimental.pallas import tpu as pltpu
from jax.experimental.pallas import tpu_sc as plsc

D = 128
LANES = 16
NUM_CORES = 2
NUM_SUBCORES = 16
NUM_WORKERS = NUM_CORES * NUM_SUBCORES
CHUNK = 200
NBUF = 2
GROUPS = ((0, 128), (128, 72))


def _sc_body(n_tokens, vocab, x_hbm, idx_hbm, tab_hbm, out_hbm, *sc):
  idx_v = sc[0:2]
  xrow = sc[2:4]
  trow = sc[4:6]
  tab_sh = sc[6]
  sem_in = sc[7:9]
  sem_g = sc[9:11]
  sem_out = sc[11:13]

  per_w = n_tokens // NUM_WORKERS
  chunks = per_w // CHUNK
  cid = lax.axis_index("c")
  sid = lax.axis_index("s")
  wid = sid * NUM_CORES + cid
  w0 = wid * per_w

  def in_copies(t, b):
    base = w0 + t * CHUNK
    return (
        pltpu.make_async_copy(
            idx_hbm.at[pl.ds(base, CHUNK)], idx_v[b], sem_in[b]),
        pltpu.make_async_copy(
            x_hbm.at[pl.ds(base, CHUNK)], xrow[b], sem_in[b]),
    )

  def gather_copies(b):
    return [
        pltpu.make_async_copy(
            tab_sh.at[idx_v[b].at[pl.ds(o, g)]],
            trow[b].at[pl.ds(o, g)],
            sem_g[b],
        )
        for (o, g) in GROUPS
    ]

  def out_copy(t, b):
    base = w0 + t * CHUNK
    return pltpu.make_async_copy(
        trow[b], out_hbm.at[pl.ds(base, CHUNK)], sem_out[b])

  for b in range(NBUF):
    for cp in in_copies(b, b):
      cp.start()

  @pl.when(sid == 0)
  def _stage():
    pltpu.sync_copy(tab_hbm, trow[0].at[pl.ds(0, vocab)])
    for j in range(D // LANES):
      trow[0][0, pl.ds(j * LANES, LANES)] = jnp.zeros((LANES,), jnp.float32)
    pltpu.sync_copy(trow[0].at[pl.ds(0, vocab)], tab_sh)

  plsc.subcore_barrier()

  def outer(tt, carry):
    for b in range(NBUF):
      t = tt * NBUF + b
      for cp in in_copies(t, b):
        cp.wait()

      @pl.when(t >= NBUF)
      def _wait_out():
        out_copy(t, b).wait()

      for cp in gather_copies(b):
        cp.start()
      for cp in gather_copies(b):
        cp.wait()

      @plsc.parallel_loop(0, CHUNK, step=1, unroll=2)
      def _add(i):
        for j in range(D // LANES):
          s = pl.ds(j * LANES, LANES)
          trow[b][i, s] = xrow[b][i, s] + trow[b][i, s]

      out_copy(t, b).start()

      @pl.when(t + NBUF < chunks)
      def _next_in():
        for cp in in_copies(t + NBUF, b):
          cp.start()

    return carry

  lax.fori_loop(0, chunks // NBUF, outer, 0)

  for b in range(NBUF):
    out_copy(chunks - NBUF + b, b).wait()


@functools.partial(jax.jit, static_argnames=())
def kernel(x, seg_idx, seg_embed):
  b, s, d = x.shape
  n = b * s
  vocab = seg_embed.shape[0]
  xf = x.reshape(n, d)
  idxf = seg_idx.reshape(n).astype(jnp.int32)
  tab = seg_embed.astype(jnp.float32)

  mesh = plsc.VectorSubcoreMesh(
      core_axis_name="c", subcore_axis_name="s",
      num_cores=NUM_CORES, num_subcores=NUM_SUBCORES,
  )
  out = pl.kernel(
      functools.partial(_sc_body, n, vocab),
      out_type=jax.ShapeDtypeStruct((n, d), jnp.float32),
      mesh=mesh,
      scratch_types=[
          pltpu.VMEM((CHUNK,), jnp.int32),
          pltpu.VMEM((CHUNK,), jnp.int32),
          pltpu.VMEM((CHUNK, D), jnp.float32),
          pltpu.VMEM((CHUNK, D), jnp.float32),
          pltpu.VMEM((CHUNK, D), jnp.float32),
          pltpu.VMEM((CHUNK, D), jnp.float32),
          pltpu.VMEM_SHARED((vocab, D), jnp.float32),
          pltpu.SemaphoreType.DMA,
          pltpu.SemaphoreType.DMA,
          pltpu.SemaphoreType.DMA,
          pltpu.SemaphoreType.DMA,
          pltpu.SemaphoreType.DMA,
          pltpu.SemaphoreType.DMA,
      ],
  )(xf, idxf, tab)
  return out.reshape(b, s, d)


# speedup vs baseline: 9.0709x; 1.2120x over previous
import functools

import jax
import jax.numpy as jnp
from jax import lax
from jax.experimental import pallas as pl
from jax.experimental.pallas import tpu as pltpu
from jax.experimental.pallas import tpu_sc as plsc

D = 128
LANES = 16
NUM_CORES = 2
NUM_SUBCORES = 16
NUM_WORKERS = NUM_CORES * NUM_SUBCORES
CHUNK = 200
NBUF = 2
GROUPS = ((0, 128), (128, 72))


def _sc_body(n_tokens, vocab, x_hbm, idx_hbm, tab_hbm, out_hbm, *sc):
  idx_v = sc[0:2]
  xrow = sc[2:4]
  stage = sc[4]
  tab_sh = sc[5]
  sem_in = sc[6:8]
  sem_g = sc[8:10]
  sem_out = sc[10:12]

  per_w = n_tokens // NUM_WORKERS
  chunks = per_w // CHUNK
  cid = lax.axis_index("c")
  sid = lax.axis_index("s")
  wid = sid * NUM_CORES + cid
  w0 = wid * per_w

  def in_copies(t, b):
    base = w0 + t * CHUNK
    return (
        pltpu.make_async_copy(
            idx_hbm.at[pl.ds(base, CHUNK)], idx_v[b], sem_in[b]),
        pltpu.make_async_copy(
            x_hbm.at[pl.ds(base, CHUNK)], xrow[b], sem_in[b]),
    )

  def gather_copies(b):
    return [
        pltpu.async_copy(
            tab_sh.at[idx_v[b].at[pl.ds(o, g)]],
            xrow[b].at[pl.ds(o, g)],
            sem_g[b],
            add=True,
        )
        for (o, g) in GROUPS
    ]

  def out_copy(t, b):
    base = w0 + t * CHUNK
    return pltpu.make_async_copy(
        xrow[b], out_hbm.at[pl.ds(base, CHUNK)], sem_out[b])

  @pl.when(sid == 0)
  def _stage():
    pltpu.sync_copy(tab_hbm, stage)
    for j in range(D // LANES):
      stage[0, pl.ds(j * LANES, LANES)] = jnp.zeros((LANES,), jnp.float32)
    pltpu.sync_copy(stage, tab_sh)

  for b in range(NBUF):
    for cp in in_copies(b, b):
      cp.start()

  plsc.subcore_barrier()

  def outer(tt, carry):
    for b in range(NBUF):
      t = tt * NBUF + b
      for cp in in_copies(t, b):
        cp.wait()

      cps = gather_copies(b)
      for cp in cps:
        cp.wait()

      out_copy(t, b).start()

      @pl.when(t + NBUF < chunks)
      def _next_in():
        out_copy(t, b).wait()
        for cp in in_copies(t + NBUF, b):
          cp.start()

    return carry

  lax.fori_loop(0, chunks // NBUF, outer, 0)

  for b in range(NBUF):
    out_copy(chunks - NBUF + b, b).wait()


@functools.partial(jax.jit, static_argnames=())
def kernel(x, seg_idx, seg_embed):
  b, s, d = x.shape
  n = b * s
  vocab = seg_embed.shape[0]
  xf = x.reshape(n, d)
  idxf = seg_idx.reshape(n).astype(jnp.int32)
  tab = seg_embed.astype(jnp.float32)

  mesh = plsc.VectorSubcoreMesh(
      core_axis_name="c", subcore_axis_name="s",
      num_cores=NUM_CORES, num_subcores=NUM_SUBCORES,
  )
  out = pl.kernel(
      functools.partial(_sc_body, n, vocab),
      out_type=jax.ShapeDtypeStruct((n, d), jnp.float32),
      mesh=mesh,
      scratch_types=[
          pltpu.VMEM((CHUNK,), jnp.int32),
          pltpu.VMEM((CHUNK,), jnp.int32),
          pltpu.VMEM((CHUNK, D), jnp.float32),
          pltpu.VMEM((CHUNK, D), jnp.float32),
          pltpu.VMEM((vocab, D), jnp.float32),
          pltpu.VMEM_SHARED((vocab, D), jnp.float32),
          pltpu.SemaphoreType.DMA,
          pltpu.SemaphoreType.DMA,
          pltpu.SemaphoreType.DMA,
          pltpu.SemaphoreType.DMA,
          pltpu.SemaphoreType.DMA,
          pltpu.SemaphoreType.DMA,
      ],
  )(xf, idxf, tab)
  return out.reshape(b, s, d)


# speedup vs baseline: 9.4624x; 1.0432x over previous
import functools

import jax
import jax.numpy as jnp
from jax import lax
from jax.experimental import pallas as pl
from jax.experimental.pallas import tpu as pltpu
from jax.experimental.pallas import tpu_sc as plsc

D = 128
LANES = 16
NUM_CORES = 2
NUM_SUBCORES = 16
NUM_WORKERS = NUM_CORES * NUM_SUBCORES
CHUNK = 200
NBUF = 4
GROUPS = ((0, 128), (128, 72))


def _sc_body(n_tokens, vocab, x_hbm, idx_hbm, tab_hbm, out_hbm, *sc):
  idx_v = sc[0:4]
  xrow = sc[4:8]
  stage = sc[8]
  tab_sh = sc[9]
  sem_in = sc[10:14]
  sem_g = sc[14]
  sem_out = sc[15:19]

  per_w = n_tokens // NUM_WORKERS
  chunks = per_w // CHUNK
  cid = lax.axis_index("c")
  sid = lax.axis_index("s")
  wid = sid * NUM_CORES + cid
  w0 = wid * per_w

  def in_copies(t, b):
    base = w0 + t * CHUNK
    return (
        pltpu.make_async_copy(
            idx_hbm.at[pl.ds(base, CHUNK)], idx_v[b], sem_in[b]),
        pltpu.make_async_copy(
            x_hbm.at[pl.ds(base, CHUNK)], xrow[b], sem_in[b]),
    )

  def gather_add(b):
    cps = [
        pltpu.async_copy(
            tab_sh.at[idx_v[b].at[pl.ds(o, g)]],
            xrow[b].at[pl.ds(o, g)],
            sem_g,
            add=True,
        )
        for (o, g) in GROUPS
    ]
    for cp in cps:
      cp.wait()

  def out_copy(t, b):
    base = w0 + t * CHUNK
    return pltpu.make_async_copy(
        xrow[b], out_hbm.at[pl.ds(base, CHUNK)], sem_out[b])

  @pl.when(sid == 0)
  def _stage():
    pltpu.sync_copy(tab_hbm, stage)
    for j in range(D // LANES):
      stage[0, pl.ds(j * LANES, LANES)] = jnp.zeros((LANES,), jnp.float32)
    pltpu.sync_copy(stage, tab_sh)

  for t in range(2):
    for cp in in_copies(t, t % NBUF):
      cp.start()

  plsc.subcore_barrier()

  def outer(tt, carry):
    for b in range(NBUF):
      u = tt * NBUF + b
      b2 = (b + 2) % NBUF
      for cp in in_copies(u, b):
        cp.wait()

      gather_add(b)
      out_copy(u, b).start()

      @pl.when(u >= 2)
      def _drain():
        out_copy(u - 2, b2).wait()

      @pl.when(u + 2 < chunks)
      def _next_in():
        for cp in in_copies(u + 2, b2):
          cp.start()

    return carry

  lax.fori_loop(0, chunks // NBUF, outer, 0)

  for t in range(chunks - 2, chunks):
    out_copy(t, t % NBUF).wait()


@functools.partial(jax.jit, static_argnames=())
def kernel(x, seg_idx, seg_embed):
  b, s, d = x.shape
  n = b * s
  vocab = seg_embed.shape[0]
  xf = x.reshape(n, d)
  idxf = seg_idx.reshape(n).astype(jnp.int32)
  tab = seg_embed.astype(jnp.float32)

  mesh = plsc.VectorSubcoreMesh(
      core_axis_name="c", subcore_axis_name="s",
      num_cores=NUM_CORES, num_subcores=NUM_SUBCORES,
  )
  out = pl.kernel(
      functools.partial(_sc_body, n, vocab),
      out_type=jax.ShapeDtypeStruct((n, d), jnp.float32),
      mesh=mesh,
      scratch_types=(
          [pltpu.VMEM((CHUNK,), jnp.int32) for _ in range(NBUF)]
          + [pltpu.VMEM((CHUNK, D), jnp.float32) for _ in range(NBUF)]
          + [
              pltpu.VMEM((vocab, D), jnp.float32),
              pltpu.VMEM_SHARED((vocab, D), jnp.float32),
          ]
          + [pltpu.SemaphoreType.DMA for _ in range(NBUF)]
          + [pltpu.SemaphoreType.DMA]
          + [pltpu.SemaphoreType.DMA for _ in range(NBUF)]
      ),
  )(xf, idxf, tab)
  return out.reshape(b, s, d)


# speedup vs baseline: 9.4677x; 1.0006x over previous
import functools

import jax
import jax.numpy as jnp
from jax import lax
from jax.experimental import pallas as pl
from jax.experimental.pallas import tpu as pltpu
from jax.experimental.pallas import tpu_sc as plsc

D = 128
LANES = 16
NUM_CORES = 2
NUM_SUBCORES = 16
NUM_WORKERS = NUM_CORES * NUM_SUBCORES
CHUNK = 200
NBUF = 4
GROUPS = ((0, 128), (128, 72))


def _sc_body(n_tokens, vocab, x_hbm, idx_hbm, tab_hbm, out_hbm, *sc):
  xrow = sc[0:4]
  idx_all = sc[4]
  tab_sh = sc[5]
  sem_idx = sc[6]
  sem_in = sc[7:11]
  sem_g = sc[11]
  sem_out = sc[12:16]

  per_w = n_tokens // NUM_WORKERS
  chunks = per_w // CHUNK
  cid = lax.axis_index("c")
  sid = lax.axis_index("s")
  wid = sid * NUM_CORES + cid
  w0 = wid * per_w

  idx_copy = pltpu.make_async_copy(
      idx_hbm.at[pl.ds(w0, per_w)], idx_all, sem_idx)

  def in_copy(t, b):
    base = w0 + t * CHUNK
    return pltpu.make_async_copy(
        x_hbm.at[pl.ds(base, CHUNK)], xrow[b], sem_in[b])

  def gather_add(t, b):
    cps = [
        pltpu.async_copy(
            tab_sh.at[idx_all.at[pl.ds(t * CHUNK + o, g)]],
            xrow[b].at[pl.ds(o, g)],
            sem_g,
            add=True,
        )
        for (o, g) in GROUPS
    ]
    for cp in cps:
      cp.wait()

  def out_copy(t, b):
    base = w0 + t * CHUNK
    return pltpu.make_async_copy(
        xrow[b], out_hbm.at[pl.ds(base, CHUNK)], sem_out[b])

  idx_copy.start()

  @pl.when(sid == 0)
  def _stage():
    pltpu.sync_copy(tab_hbm, xrow[0].at[pl.ds(0, vocab)])
    for j in range(D // LANES):
      xrow[0][0, pl.ds(j * LANES, LANES)] = jnp.zeros((LANES,), jnp.float32)
    pltpu.sync_copy(xrow[0].at[pl.ds(0, vocab)], tab_sh)

  for t in range(2):
    in_copy(t, t % NBUF).start()

  plsc.subcore_barrier()
  idx_copy.wait()

  def outer(tt, carry):
    for b in range(NBUF):
      u = tt * NBUF + b
      b2 = (b + 2) % NBUF
      in_copy(u, b).wait()

      gather_add(u, b)
      out_copy(u, b).start()

      @pl.when(u >= 2)
      def _drain():
        out_copy(u - 2, b2).wait()

      @pl.when(u + 2 < chunks)
      def _next_in():
        in_copy(u + 2, b2).start()

    return carry

  lax.fori_loop(0, chunks // NBUF, outer, 0)

  for t in range(chunks - 2, chunks):
    out_copy(t, t % NBUF).wait()


@functools.partial(jax.jit, static_argnames=())
def kernel(x, seg_idx, seg_embed):
  b, s, d = x.shape
  n = b * s
  vocab = seg_embed.shape[0]
  xf = x.reshape(n, d)
  idxf = seg_idx.reshape(n).astype(jnp.int32)
  tab = seg_embed.astype(jnp.float32)

  mesh = plsc.VectorSubcoreMesh(
      core_axis_name="c", subcore_axis_name="s",
      num_cores=NUM_CORES, num_subcores=NUM_SUBCORES,
  )
  out = pl.kernel(
      functools.partial(_sc_body, n, vocab),
      out_type=jax.ShapeDtypeStruct((n, d), jnp.float32),
      mesh=mesh,
      scratch_types=(
          [pltpu.VMEM((CHUNK, D), jnp.float32) for _ in range(NBUF)]
          + [
              pltpu.VMEM((n // NUM_WORKERS,), jnp.int32),
              pltpu.VMEM_SHARED((vocab, D), jnp.float32),
          ]
          + [pltpu.SemaphoreType.DMA for _ in range(2 * NBUF + 2)]
      ),
  )(xf, idxf, tab)
  return out.reshape(b, s, d)
